# baseline (speedup 1.0000x reference)
import functools

import jax
import jax.numpy as jnp
from jax import lax
from jax.experimental import pallas as pl
from jax.experimental.pallas import tpu as pltpu
from jax.experimental.pallas import tpu_sc as plsc

B, S, D = 16384, 6, 512
NC, NS = 2, 16
NW = NC * NS
BPW = B // NW
CB = 8
NCH = BPW // CB


@functools.partial(
    pl.kernel,
    out_type=jax.ShapeDtypeStruct((B, S, D), jnp.float32),
    mesh=plsc.VectorSubcoreMesh(core_axis_name="c", subcore_axis_name="s"),
    scratch_types=[
        pltpu.VMEM((16,), jnp.int32),
        pltpu.VMEM((CB, S, D), jnp.float32),
        pltpu.VMEM((CB, S, D), jnp.float32),
        pltpu.SemaphoreType.DMA,
        pltpu.SemaphoreType.DMA,
        pltpu.SemaphoreType.DMA,
        pltpu.SemaphoreType.DMA,
    ],
    compiler_params=pltpu.CompilerParams(
        use_tc_tiling_on_sc=True, needs_layout_passes=False),
)
def _shuffle_chunks(x_hbm, tab_hbm, out_hbm, tab_v, buf0, buf1,
                    g0, g1, w0, w1):
    wid = lax.axis_index("s") * NC + lax.axis_index("c")
    b0 = wid * BPW

    pltpu.sync_copy(tab_hbm, tab_v)
    tab_vec = tab_v[...]
    iota = lax.broadcasted_iota(jnp.int32, (16,), 0)
    pis = [jnp.max(jnp.where(iota == i, tab_vec, 0)) for i in range(S)]

    buf = (buf0, buf1)
    gsem = (g0, g1)
    wsem = (w0, w1)

    def gathers(c):
        b = c % 2
        return [
            pltpu.async_copy(
                x_hbm.at[pl.ds(b0 + c * CB, CB), pl.ds(pis[i], 1), :],
                buf[b].at[:, pl.ds(i, 1), :],
                gsem[b])
            for i in range(S)
        ]

    def write(c):
        b = c % 2
        return pltpu.async_copy(
            buf[b], out_hbm.at[pl.ds(b0 + c * CB, CB), :, :], wsem[b])

    gh = [None, None]
    wh = [None, None]
    gh[0] = gathers(0)
    for c in range(NCH):
        b = c % 2
        if c + 1 < NCH:
            nb = (c + 1) % 2
            if wh[nb] is not None:
                wh[nb].wait()
            gh[nb] = gathers(c + 1)
        for h in gh[b]:
            h.wait()
        wh[b] = write(c)
    wh[0].wait()
    wh[1].wait()


def kernel(x, index):
    tab16 = jnp.zeros((16,), jnp.int32).at[:S].set(index.astype(jnp.int32))
    return _shuffle_chunks(x, tab16)
